# baseline (speedup 1.0000x reference)
import functools

import jax
import jax.numpy as jnp
from jax import lax
from jax.experimental import pallas as pl
from jax.experimental.pallas import tpu as pltpu
from jax.experimental.pallas import tpu_sc as plsc

B = 16384
D = 64
H = 128

NC = 2
NS = 16
NW = NC * NS
BPW = B // NW


def _sc_gather(user, movie, ut, mt):
  mesh = plsc.VectorSubcoreMesh(core_axis_name="c", subcore_axis_name="s")

  @functools.partial(
      pl.kernel,
      mesh=mesh,
      out_type=[
          jax.ShapeDtypeStruct((B, D), jnp.float32),
          jax.ShapeDtypeStruct((B, D), jnp.float32),
      ],
      scratch_types=[
          pltpu.VMEM((BPW,), jnp.int32),
          pltpu.VMEM((BPW,), jnp.int32),
          pltpu.VMEM((BPW, D), jnp.float32),
          pltpu.VMEM((BPW, D), jnp.float32),
          pltpu.SemaphoreType.DMA,
          pltpu.SemaphoreType.DMA,
      ],
      compiler_params=pltpu.CompilerParams(use_tc_tiling_on_sc=False),
  )
  def k(ur_hbm, mr_hbm, ut_hbm, mt_hbm, xu_hbm, xv_hbm,
        idx_u, idx_m, rows_u, rows_m, sem_u, sem_m):
    wid = lax.axis_index("s") * NC + lax.axis_index("c")
    base = wid * BPW
    pltpu.sync_copy(ur_hbm.at[pl.ds(base, BPW)], idx_u)
    pltpu.sync_copy(mr_hbm.at[pl.ds(base, BPW)], idx_m)
    cu = pltpu.async_copy(ut_hbm.at[idx_u], rows_u, sem_u)
    cm = pltpu.async_copy(mt_hbm.at[idx_m], rows_m, sem_m)
    cu.wait()
    cm.wait()
    pltpu.sync_copy(rows_u, xu_hbm.at[pl.ds(base, BPW)])
    pltpu.sync_copy(rows_m, xv_hbm.at[pl.ds(base, BPW)])

  return k(user, movie, ut, mt)


BLK = 2048


def _mlp_body(xu_ref, xv_ref, w1a_ref, w1b_ref, b1_ref, w2_ref, b2_ref,
              out_ref):
  h = jnp.dot(xu_ref[...], w1a_ref[...], preferred_element_type=jnp.float32)
  h = h + jnp.dot(xv_ref[...], w1b_ref[...],
                  preferred_element_type=jnp.float32)
  h = jnp.maximum(h + b1_ref[...], 0.0)
  o = jnp.dot(h, w2_ref[...], preferred_element_type=jnp.float32)
  out_ref[...] = o[:, 0] + b2_ref[0, 0]


def _mlp(xu, xv, w1a, w1b, b1r, w2, b2r):
  return pl.pallas_call(
      _mlp_body,
      grid=(B // BLK,),
      in_specs=[
          pl.BlockSpec((BLK, D), lambda i: (i, 0)),
          pl.BlockSpec((BLK, D), lambda i: (i, 0)),
          pl.BlockSpec((D, H), lambda i: (0, 0)),
          pl.BlockSpec((D, H), lambda i: (0, 0)),
          pl.BlockSpec((1, H), lambda i: (0, 0)),
          pl.BlockSpec((H, 1), lambda i: (0, 0)),
          pl.BlockSpec((1, 1), lambda i: (0, 0)),
      ],
      out_specs=pl.BlockSpec((BLK,), lambda i: (i,)),
      out_shape=jax.ShapeDtypeStruct((B,), jnp.float32),
  )(xu, xv, w1a, w1b, b1r, w2, b2r)


def kernel(user, movie, user_table, movie_table, W1, b1, W2, b2):
  xu, xv = _sc_gather(user, movie, user_table, movie_table)
  w1a = W1[:D]
  w1b = W1[D:]
  b1r = b1.reshape(1, H)
  b2r = b2.reshape(1, 1)
  return _mlp(xu, xv, w1a, w1b, b1r, W2, b2r)
